# Initial kernel scaffold; baseline (speedup 1.0000x reference)
import jax
import jax.numpy as jnp
from jax.experimental import pallas as pl


def kernel(species, coordinates):
    raise NotImplementedError("write your pallas kernel here")


# speedup vs baseline: 24.7915x; 24.7915x over previous
import math

import jax
import jax.numpy as jnp
import numpy as np
from jax.experimental import pallas as pl

_Rcr = 5.2
_Rca = 3.5
_EtaR = 16.0
_EtaA = 8.0
_ShfR = np.array([0.9, 1.16875, 1.4375, 1.70625, 1.975, 2.24375, 2.5125,
                  2.78125, 3.05, 3.31875, 3.5875, 3.85625, 4.125, 4.39375,
                  4.6625, 4.93125], dtype=np.float32)
_ShfA = np.array([0.9, 1.55, 2.2, 2.85], dtype=np.float32)
_ShfZ = np.array([0.19634954, 0.58904862, 0.9817477, 1.3744468, 1.7671459,
                  2.1598449, 2.552544, 2.9452431], dtype=np.float32)
_NS = 4
_NSP = 10
_C, _A = 32, 32
_RSUB = 16
_ASUB = 32

_COSZ = np.cos(_ShfZ.astype(np.float64)).astype(np.float32)
_SINZ = np.sin(_ShfZ.astype(np.float64)).astype(np.float32)

_HIGH = jax.lax.Precision.HIGHEST


def _aev_body(sp_ref, ct_ref, out_ref):
    sp = sp_ref[0, 0, :]
    cx = ct_ref[0, 0, :]
    cy = ct_ref[0, 1, :]
    cz = ct_ref[0, 2, :]

    dx = cx[:, None] - cx[None, :]
    dy = cy[:, None] - cy[None, :]
    dz = cz[:, None] - cz[None, :]
    d2 = dx * dx + dy * dy + dz * dz
    dist = jnp.sqrt(d2)

    ii = jax.lax.broadcasted_iota(jnp.int32, (_A, _A), 0)
    jj = jax.lax.broadcasted_iota(jnp.int32, (_A, _A), 1)
    offdiag = ii != jj

    fcr = 0.5 * jnp.cos(dist * (math.pi / _Rcr)) + 0.5
    mr = (dist <= _Rcr) & offdiag
    rtm = jnp.where(mr, 0.25 * fcr, 0.0)
    rt_blocks = []
    for shfr in _ShfR.tolist():
        w = dist - shfr
        rt_blocks.append(jnp.exp(-_EtaR * (w * w)) * rtm)
    rt = jnp.stack(rt_blocks, axis=1)
    eyeS = (sp[:, None] ==
            jax.lax.broadcasted_iota(jnp.int32, (_A, _NS), 1)).astype(jnp.float32)
    rad = jax.lax.dot_general(rt.reshape(_A * _RSUB, _A), eyeS,
                              (((1,), (0,)), ((), ())),
                              precision=_HIGH,
                              preferred_element_type=jnp.float32)
    rad = rad.reshape(_A, _RSUB, _NS)
    rad = jnp.swapaxes(rad, 1, 2).reshape(_A, _NS * _RSUB)

    fca = 0.5 * jnp.cos(dist * (math.pi / _Rca)) + 0.5
    ma = (dist <= _Rca) & offdiag
    fcm = jnp.where(ma, fca, 0.0)
    ds = dist + (~offdiag).astype(jnp.float32)

    dots = (dx[:, :, None] * dx[:, None, :] +
            dy[:, :, None] * dy[:, None, :] +
            dz[:, :, None] * dz[:, None, :])
    ccd = 0.95 * dots / (ds[:, :, None] * ds[:, None, :])
    avgd = 0.5 * (dist[:, :, None] + dist[:, None, :])
    fcpd = fcm[:, :, None] * fcm[:, None, :]

    cc = ccd.reshape(_A, _A * _A)
    avg = avgd.reshape(_A, _A * _A)
    fcp = fcpd.reshape(_A, _A * _A)
    u = 0.5 * cc
    v = 0.5 * jnp.sqrt(1.0 - cc * cc)

    t32 = []
    for z in range(8):
        t = 0.5 + _COSZ[z].item() * u + _SINZ[z].item() * v
        t = t * t
        t = t * t
        t = t * t
        t = t * t
        t = t * t
        t32.append(t)
    blocks = []
    for s in range(4):
        w = avg - _ShfA[s].item()
        g = jnp.exp(-_EtaA * (w * w)) * fcp
        for z in range(8):
            blocks.append(g * t32[z])
    tmat = jnp.stack(blocks, axis=1)
    tmat = tmat.reshape(_A * _ASUB, _A * _A)

    spj = jnp.broadcast_to(sp[:, None], (_A, _A))
    spk = jnp.broadcast_to(sp[None, :], (_A, _A))
    mn = jnp.minimum(spj, spk)
    mx = jnp.maximum(spj, spk)
    pidx = mn * _NS - (mn * (mn + 1)) // 2 + mx
    p3 = jax.lax.broadcasted_iota(jnp.int32, (_NSP, _A, _A), 0)
    j3 = jax.lax.broadcasted_iota(jnp.int32, (_NSP, _A, _A), 1)
    k3 = jax.lax.broadcasted_iota(jnp.int32, (_NSP, _A, _A), 2)
    pidx3 = jnp.broadcast_to(pidx[None, :, :], (_NSP, _A, _A))
    pm3 = ((pidx3 == p3) & (j3 != k3)).astype(jnp.float32)
    pmatT = pm3.reshape(_NSP, _A * _A)

    ang = jax.lax.dot_general(tmat, pmatT, (((1,), (1,)), ((), ())),
                              precision=_HIGH,
                              preferred_element_type=jnp.float32)
    ang = ang.reshape(_A, _ASUB, _NSP)
    ang = jnp.swapaxes(ang, 1, 2).reshape(_A, _NSP * _ASUB)

    out_ref[0] = jnp.concatenate([rad, ang], axis=-1)


def kernel(species, coordinates):
    sp3 = species.reshape(_C, 1, _A).astype(jnp.int32)
    ct = jnp.transpose(coordinates, (0, 2, 1))
    out = pl.pallas_call(
        _aev_body,
        grid=(_C,),
        in_specs=[
            pl.BlockSpec((1, 1, _A), lambda m: (m, 0, 0)),
            pl.BlockSpec((1, 3, _A), lambda m: (m, 0, 0)),
        ],
        out_specs=pl.BlockSpec((1, _A, _NS * _RSUB + _NSP * _ASUB),
                               lambda m: (m, 0, 0)),
        out_shape=jax.ShapeDtypeStruct(
            (_C, _A, _NS * _RSUB + _NSP * _ASUB), jnp.float32),
    )(sp3, ct)
    return out
